# baseline (speedup 1.0000x reference)
import jax
import jax.numpy as jnp
from jax import lax
from jax.experimental import pallas as pl
from jax.experimental.pallas import tpu as pltpu
from jax.experimental.pallas import tpu_sc as plsc

_NUM_CLASSES = 1000000
_HIDDEN = 64
_BATCH = 16384

_NC = 2
_NS = 16
_LANES = 16
_NW = _NC * _NS
_BPW = _BATCH // _NW
_CHUNK = 128
_NCHUNK = _BPW // _CHUNK


def _emb_kernel(labels_hbm, drops_hbm, table_hbm, out_hbm,
                drops_v, idx_v, dr_v, rows_v, gsem, dsem):
    wid = lax.axis_index("s") * _NC + lax.axis_index("c")
    base = wid * _BPW

    pltpu.sync_copy(drops_hbm.at[pl.ds(base, _BPW)], drops_v)

    pltpu.sync_copy(labels_hbm.at[wid], idx_v)

    pltpu.async_copy(table_hbm.at[pl.ds(_NUM_CLASSES, 1)], dr_v, dsem).wait()

    copies = [
        pltpu.async_copy(table_hbm.at[idx_v.at[j]], rows_v.at[j], gsem)
        for j in range(_NCHUNK)
    ]
    for cp in copies:
        cp.wait()

    drj = [dr_v[0, pl.ds(j * _LANES, _LANES)] for j in range(_HIDDEN // _LANES)]

    def fix(g, carry):
        dvec = drops_v[pl.ds(g * _LANES, _LANES)]
        for k in range(_LANES):
            @pl.when(dvec[k] == 1)
            def _():
                i = g * _LANES + k
                row = rows_v.at[i // _CHUNK].at[lax.rem(i, _CHUNK)]
                for j in range(_HIDDEN // _LANES):
                    row[pl.ds(j * _LANES, _LANES)] = drj[j]
        return carry

    lax.fori_loop(0, _BPW // _LANES, fix, 0)

    for j in range(_NCHUNK):
        pltpu.sync_copy(rows_v.at[j],
                        out_hbm.at[pl.ds(base + j * _CHUNK, _CHUNK)])


@jax.jit
def _embed(labels, drops, table):
    mesh = plsc.VectorSubcoreMesh(core_axis_name="c", subcore_axis_name="s")
    return pl.kernel(
        _emb_kernel,
        mesh=mesh,
        out_type=jax.ShapeDtypeStruct((_BATCH, _HIDDEN), jnp.float32),
        scratch_types=[
            pltpu.VMEM((_BPW,), jnp.int32),
            pltpu.VMEM((_NCHUNK, _CHUNK), jnp.int32),
            pltpu.VMEM((1, _HIDDEN), jnp.float32),
            pltpu.VMEM((_NCHUNK, _CHUNK, _HIDDEN), jnp.float32),
            pltpu.SemaphoreType.DMA,
            pltpu.SemaphoreType.DMA,
        ],
        compiler_params=pltpu.CompilerParams(use_tc_tiling_on_sc=False),
    )(labels, drops, table)


def kernel(class_labels, train, force_drop_ids, table):
    del train
    return _embed(class_labels.astype(jnp.int32).reshape(_NW, _NCHUNK, _CHUNK),
                  force_drop_ids.astype(jnp.int32), table)
